# baseline (speedup 1.0000x reference)
import functools

import numpy as np
import jax
import jax.numpy as jnp
from jax import lax
from jax.experimental import pallas as pl
from jax.experimental.pallas import tpu as pltpu
from jax.experimental.pallas import tpu_sc as plsc

_N = 64
_MEM = 6
_B = 1024
_L = 16
_NW = 32
_BPW = _B // _NW
_STEPS = 128
_OUT_STEPS = 65
_ROWLEN = _OUT_STEPS * _N
_TBL = _L * _ROWLEN
_PSTR = _L + 1
_XW = _STEPS * _BPW
_CLAMP = 50.0
_INIT = 20.0


def _branch_sign_structure():
    gm = np.array([[1, 1, 1, 1, 0, 0, 1], [1, 0, 1, 1, 0, 1, 1]], dtype=np.int64)
    s = np.arange(_N)[:, None]
    b = np.arange(2)[None, :]
    p = 2 * (s % (_N // 2)) + b
    u = np.broadcast_to(s >> (_MEM - 1), p.shape)
    bits = np.zeros((_N, 2, _MEM + 1), dtype=np.int64)
    bits[:, :, 0] = u
    for j in range(_MEM):
        bits[:, :, j + 1] = (p >> (_MEM - 1 - j)) & 1
    c = np.einsum('rk,sbk->rsb', gm, bits) % 2
    signs = 1.0 - 2.0 * c
    s00, s10 = signs[0, :, 0], signs[1, :, 0]
    assert np.all(signs[0, :, 1] == -s00) and np.all(signs[1, :, 1] == -s10)
    use_sum = [bool(s00[i] == s10[i]) for i in range(_N)]
    positive = [bool(s00[i] > 0) for i in range(_N)]
    return use_sum, positive


_USE_SUM, _POSITIVE = _branch_sign_structure()


def _acs_step(x_vm, w_vm, src, dst, tb, col, choff, wr, giota,
              weighted, emit):
    x0 = x_vm[pl.ds(col * _BPW + choff, _L)]
    x1 = x_vm[pl.ds(col * _BPW + _BPW + choff, _L)]
    asum = x0 + x1
    adif = x0 - x1
    if weighted:
        wvec = [w_vm[pl.ds(wr * _N + g * _L, _L)] for g in range(_N // _L)]
        ws = [wvec[s >> 4][s & 15] for s in range(_N)]
    sums = [None] * 8
    for m in range(_N // 2):
        p0 = src[pl.ds(2 * m * _PSTR, _L)]
        p1 = src[pl.ds((2 * m + 1) * _PSTR, _L)]
        for s in (m, m + _N // 2):
            sel = asum if _USE_SUM[s] else adif
            t = ws[s] * sel if weighted else sel
            if _POSITIVE[s]:
                o = jnp.maximum(p0 + t, p1 - t)
            else:
                o = jnp.maximum(p0 - t, p1 + t)
            dst[pl.ds(s * _PSTR, _L)] = o
            j = s & 7
            sums[j] = o if sums[j] is None else sums[j] + o
    mean = (((sums[0] + sums[1]) + (sums[2] + sums[3]))
            + ((sums[4] + sums[5]) + (sums[6] + sums[7]))) * (1.0 / _N)
    for s in range(_N):
        v = dst[pl.ds(s * _PSTR, _L)] - mean
        dst[pl.ds(s * _PSTR, _L)] = jnp.minimum(
            jnp.maximum(v, -_CLAMP), _CLAMP)
    if emit:
        rbase = wr * _N
        for j in range(_L):
            for g in range(_N // _L):
                col16 = plsc.load_gather(dst, [giota + (g * _L * _PSTR + j)])
                tb[j, pl.ds(rbase + g * _L, _L)] = col16


def _sc_decode(x_in, w_in):
    mesh = plsc.VectorSubcoreMesh(core_axis_name="c", subcore_axis_name="s")

    @functools.partial(
        pl.kernel,
        mesh=mesh,
        compiler_params=pltpu.CompilerParams(needs_layout_passes=False),
        out_type=jax.ShapeDtypeStruct((_B, _ROWLEN), jnp.float32),
        scratch_types=[
            pltpu.VMEM((_XW,), jnp.float32),
            pltpu.VMEM((_OUT_STEPS * _N,), jnp.float32),
            pltpu.VMEM((_N * _PSTR,), jnp.float32),
            pltpu.VMEM((_N * _PSTR,), jnp.float32),
            pltpu.VMEM((_L, _ROWLEN), jnp.float32),
            pltpu.SemaphoreType.DMA,
        ],
    )
    def k(x_hbm, w_hbm, out_hbm, x_vm, w_vm, pa, pb, tb, sem_t):
        wid = lax.axis_index("s") * 2 + lax.axis_index("c")
        pltpu.sync_copy(
            x_hbm.at[pl.ds(pl.multiple_of(wid * _XW, _XW), _XW)], x_vm)
        pltpu.sync_copy(w_hbm, w_vm)
        giota = lax.iota(jnp.int32, _L) * _PSTR
        init = jnp.full((_L,), _INIT, jnp.float32)
        zero = jnp.zeros((_L,), jnp.float32)

        def out_blk(ch):
            roff = pl.multiple_of((2 * wid + ch) * _L, _L)
            return out_hbm.at[pl.ds(roff, _L), :]

        def one_pass(ch, carry):
            choff = ch * _L
            pa[pl.ds(0, _L)] = init
            for s in range(1, _N):
                pa[pl.ds(s * _PSTR, _L)] = zero

            def ph1(kk, c2):
                _acs_step(x_vm, w_vm, pa, pb, tb, 4 * kk, choff, 0,
                          giota, False, False)
                _acs_step(x_vm, w_vm, pb, pa, tb, 4 * kk + 2, choff, 0,
                          giota, False, False)
                return c2

            lax.fori_loop(0, 31, ph1, 0)
            _acs_step(x_vm, w_vm, pa, pb, tb, 124, choff, 0,
                      giota, False, False)
            @pl.when(ch > 0)
            def _():
                pltpu.make_async_copy(tb, out_blk(ch - 1), sem_t).wait()
            _acs_step(x_vm, w_vm, pb, pa, tb, 126, choff, 0,
                      giota, True, True)

            def ph2(kk, c2):
                _acs_step(x_vm, w_vm, pa, pb, tb, 4 * kk, choff,
                          2 * kk + 1, giota, True, True)
                _acs_step(x_vm, w_vm, pb, pa, tb, 4 * kk + 2, choff,
                          2 * kk + 2, giota, True, True)
                return c2

            lax.fori_loop(0, 32, ph2, 0)
            pltpu.async_copy(tb, out_blk(ch), sem_t)
            return carry

        lax.fori_loop(0, 2, one_pass, 0)
        pltpu.make_async_copy(tb, out_blk(1), sem_t).wait()

    return k(x_in, w_in)


def kernel(x, weights):
    x_in = (x.T.reshape(_STEPS, _NW, _BPW)
            .transpose(1, 0, 2).reshape(_NW * _STEPS * _BPW))
    w_in = weights[_STEPS - _OUT_STEPS:_STEPS].reshape(_OUT_STEPS * _N)
    return _sc_decode(x_in, w_in)


# speedup vs baseline: 1.0101x; 1.0101x over previous
import functools

import numpy as np
import jax
import jax.numpy as jnp
from jax import lax
from jax.experimental import pallas as pl
from jax.experimental.pallas import tpu as pltpu
from jax.experimental.pallas import tpu_sc as plsc

_N = 64
_MEM = 6
_B = 1024
_L = 16
_NW = 32
_BPW = _B // _NW
_STEPS = 128
_OUT_STEPS = 65
_ROWLEN = _OUT_STEPS * _N
_TBL = _L * _ROWLEN
_PSTR = _L + 1
_XW = _STEPS * _BPW
_CLAMP = 50.0
_INIT = 20.0


def _branch_sign_structure():
    gm = np.array([[1, 1, 1, 1, 0, 0, 1], [1, 0, 1, 1, 0, 1, 1]], dtype=np.int64)
    s = np.arange(_N)[:, None]
    b = np.arange(2)[None, :]
    p = 2 * (s % (_N // 2)) + b
    u = np.broadcast_to(s >> (_MEM - 1), p.shape)
    bits = np.zeros((_N, 2, _MEM + 1), dtype=np.int64)
    bits[:, :, 0] = u
    for j in range(_MEM):
        bits[:, :, j + 1] = (p >> (_MEM - 1 - j)) & 1
    c = np.einsum('rk,sbk->rsb', gm, bits) % 2
    signs = 1.0 - 2.0 * c
    s00, s10 = signs[0, :, 0], signs[1, :, 0]
    assert np.all(signs[0, :, 1] == -s00) and np.all(signs[1, :, 1] == -s10)
    use_sum = [bool(s00[i] == s10[i]) for i in range(_N)]
    positive = [bool(s00[i] > 0) for i in range(_N)]
    return use_sum, positive


_USE_SUM, _POSITIVE = _branch_sign_structure()


def _acs_step(x_vm, w_vm, src, dst, tb, col, choff, wr, giota,
              weighted, emit):
    x0 = x_vm[pl.ds(col * _BPW + choff, _L)]
    x1 = x_vm[pl.ds(col * _BPW + _BPW + choff, _L)]
    asum = x0 + x1
    adif = x0 - x1
    if weighted:
        wvec = [w_vm[pl.ds(wr * _N + g * _L, _L)] for g in range(_N // _L)]
        ws = [wvec[s >> 4][s & 15] for s in range(_N)]
    sums = [None, None, None, None]
    for m in range(_N // 2):
        p0 = src[pl.ds(2 * m * _PSTR, _L)]
        p1 = src[pl.ds((2 * m + 1) * _PSTR, _L)]
        for s in (m, m + _N // 2):
            sel = asum if _USE_SUM[s] else adif
            t = ws[s] * sel if weighted else sel
            if _POSITIVE[s]:
                o = jnp.maximum(p0 + t, p1 - t)
            else:
                o = jnp.maximum(p0 - t, p1 + t)
            dst[pl.ds(s * _PSTR, _L)] = o
            j = s & 3
            sums[j] = o if sums[j] is None else sums[j] + o
    mean = ((sums[0] + sums[1]) + (sums[2] + sums[3])) * (1.0 / _N)
    for s in range(_N):
        v = dst[pl.ds(s * _PSTR, _L)] - mean
        dst[pl.ds(s * _PSTR, _L)] = jnp.minimum(
            jnp.maximum(v, -_CLAMP), _CLAMP)
    if emit:
        rbase = wr * _N
        for j in range(_L):
            for g in range(_N // _L):
                col16 = plsc.load_gather(dst, [giota + (g * _L * _PSTR + j)])
                tb[j, pl.ds(rbase + g * _L, _L)] = col16


def _sc_decode(x_in, w_in):
    mesh = plsc.VectorSubcoreMesh(core_axis_name="c", subcore_axis_name="s")

    @functools.partial(
        pl.kernel,
        mesh=mesh,
        compiler_params=pltpu.CompilerParams(needs_layout_passes=False),
        out_type=jax.ShapeDtypeStruct((_B, _ROWLEN), jnp.float32),
        scratch_types=[
            pltpu.VMEM((_XW,), jnp.float32),
            pltpu.VMEM((_OUT_STEPS * _N,), jnp.float32),
            pltpu.VMEM((_N * _PSTR,), jnp.float32),
            pltpu.VMEM((_N * _PSTR,), jnp.float32),
            pltpu.VMEM((_L, _ROWLEN), jnp.float32),
            pltpu.SemaphoreType.DMA,
        ],
    )
    def k(x_hbm, w_hbm, out_hbm, x_vm, w_vm, pa, pb, tb, sem_t):
        wid = lax.axis_index("s") * 2 + lax.axis_index("c")
        pltpu.sync_copy(
            x_hbm.at[pl.ds(pl.multiple_of(wid * _XW, _XW), _XW)], x_vm)
        pltpu.sync_copy(w_hbm, w_vm)
        giota = lax.iota(jnp.int32, _L) * _PSTR
        init = jnp.full((_L,), _INIT, jnp.float32)
        zero = jnp.zeros((_L,), jnp.float32)

        def out_blk(ch):
            roff = pl.multiple_of((2 * wid + ch) * _L, _L)
            return out_hbm.at[pl.ds(roff, _L), :]

        def one_pass(ch, carry):
            choff = ch * _L
            pa[pl.ds(0, _L)] = init
            for s in range(1, _N):
                pa[pl.ds(s * _PSTR, _L)] = zero

            def ph1(kk, c2):
                _acs_step(x_vm, w_vm, pa, pb, tb, 4 * kk, choff, 0,
                          giota, False, False)
                _acs_step(x_vm, w_vm, pb, pa, tb, 4 * kk + 2, choff, 0,
                          giota, False, False)
                return c2

            lax.fori_loop(0, 31, ph1, 0)
            _acs_step(x_vm, w_vm, pa, pb, tb, 124, choff, 0,
                      giota, False, False)
            @pl.when(ch > 0)
            def _():
                pltpu.make_async_copy(tb, out_blk(ch - 1), sem_t).wait()
            _acs_step(x_vm, w_vm, pb, pa, tb, 126, choff, 0,
                      giota, True, True)

            def ph2(kk, c2):
                _acs_step(x_vm, w_vm, pa, pb, tb, 4 * kk, choff,
                          2 * kk + 1, giota, True, True)
                _acs_step(x_vm, w_vm, pb, pa, tb, 4 * kk + 2, choff,
                          2 * kk + 2, giota, True, True)
                return c2

            lax.fori_loop(0, 32, ph2, 0)
            pltpu.async_copy(tb, out_blk(ch), sem_t)
            return carry

        lax.fori_loop(0, 2, one_pass, 0)
        pltpu.make_async_copy(tb, out_blk(1), sem_t).wait()

    return k(x_in, w_in)


def kernel(x, weights):
    x_in = (x.T.reshape(_STEPS, _NW, _BPW)
            .transpose(1, 0, 2).reshape(_NW * _STEPS * _BPW))
    w_in = weights[_STEPS - _OUT_STEPS:_STEPS].reshape(_OUT_STEPS * _N)
    return _sc_decode(x_in, w_in)
